# baseline (speedup 1.0000x reference)
import functools

import jax
import jax.numpy as jnp
from jax import lax
from jax.experimental import pallas as pl
from jax.experimental.pallas import tpu as pltpu
from jax.experimental.pallas import tpu_sc as plsc

N = 10000
D = 128
NC = 2
NS = 16
NW = NC * NS
CHUNK = 128
BLK = 128
SB = 16
C0 = 80
C1 = 80
TBLK = NS * (C0 + C1)
EPAD = TBLK * BLK
CPW = EPAD // NW // CHUNK
NPAD = 10112
TRASH = N
RPT = NPAD // NS
OPT = 624
OTAIL = N - NS * OPT

_MESH = plsc.VectorSubcoreMesh(core_axis_name="c", subcore_axis_name="s")


@functools.partial(
    pl.kernel,
    out_type=jax.ShapeDtypeStruct((NPAD,), jnp.float32),
    mesh=_MESH,
    scratch_types=[
        pltpu.VMEM((CPW, CHUNK), jnp.int32),
        pltpu.VMEM((CPW, CHUNK), jnp.int32),
        pltpu.VMEM((CHUNK,), jnp.float32),
        pltpu.VMEM((640,), jnp.float32),
        pltpu.VMEM_SHARED((NPAD,), jnp.float32),
        pltpu.SemaphoreType.DMA,
    ],
)
def _deg_kernel(col_hbm, deg_out, colva, colvb, ones, stage, dacc, sem):
    c = lax.axis_index("c")
    s = lax.axis_index("s")

    @pl.when(c == 0)
    def _():
        for i in range(640 // 16):
            stage[pl.ds(i * 16, 16)] = jnp.zeros((16,), jnp.float32)
        for i in range(CHUNK // 16):
            ones[pl.ds(i * 16, 16)] = jnp.ones((16,), jnp.float32)
        pltpu.sync_copy(stage.at[pl.ds(0, RPT)], dacc.at[pl.ds(s * RPT, RPT)])
        pltpu.sync_copy(col_hbm.at[2 * s], colva)
        pltpu.sync_copy(col_hbm.at[2 * s + 1], colvb)
        plsc.subcore_barrier()

        @pl.loop(0, CPW)
        def _(j):
            pltpu.async_copy(ones, dacc.at[colva.at[j]], sem, add=True)

        @pl.loop(0, CPW)
        def _(j):
            pltpu.async_copy(ones, dacc.at[colvb.at[j]], sem, add=True)

        @pl.loop(0, 2 * CPW)
        def _(j):
            pltpu.make_async_copy(ones, dacc.at[colva.at[0]], sem).wait()

        plsc.subcore_barrier()
        pltpu.sync_copy(dacc.at[pl.ds(s * RPT, RPT)], stage.at[pl.ds(0, RPT)])
        pltpu.sync_copy(stage.at[pl.ds(0, RPT)],
                        deg_out.at[pl.ds(s * RPT, RPT)])


@functools.partial(
    pl.kernel,
    out_type=jax.ShapeDtypeStruct((NC, N, D), jnp.float32),
    mesh=_MESH,
    scratch_types=[
        pltpu.VMEM((SB, BLK), jnp.int32),
        pltpu.VMEM((SB, BLK), jnp.int32),
        pltpu.VMEM((BLK, D), jnp.float32),
        pltpu.VMEM((BLK, D), jnp.float32),
        pltpu.VMEM_SHARED((NPAD, D), jnp.float32),
        pltpu.SemaphoreType.DMA,
        pltpu.SemaphoreType.DMA,
    ],
)
def _msg_kernel(hs_hbm, row_hbm, col_hbm, m_out, rowv, colv, g0, g1, macc,
                sem0, sem1):
    c = lax.axis_index("c")
    s = lax.axis_index("s")
    base = jnp.where(c == 0, s * C0, NS * C0 + s * C1)
    nsb = jnp.where(c == 0, C0 // SB, C1 // SB)

    @pl.loop(0, BLK)
    def _(i):
        for l in range(D // 16):
            g0[i, pl.ds(l * 16, 16)] = jnp.zeros((16,), jnp.float32)

    for t in range(RPT // BLK):
        pltpu.sync_copy(g0, macc.at[pl.ds(s * RPT + t * BLK, BLK)])
    rem = RPT - (RPT // BLK) * BLK
    pltpu.sync_copy(g0.at[pl.ds(0, rem)],
                    macc.at[pl.ds(s * RPT + RPT - rem, rem)])
    plsc.subcore_barrier()

    @pl.loop(0, nsb)
    def _(sb):
        off = pl.multiple_of(base + sb * SB, 8)
        pltpu.sync_copy(row_hbm.at[pl.ds(off, SB)], rowv)
        pltpu.sync_copy(col_hbm.at[pl.ds(off, SB)], colv)

        @pl.loop(0, SB // 2)
        def _(j):
            a0 = pltpu.async_copy(hs_hbm.at[rowv.at[2 * j]], g0, sem0)
            a1 = pltpu.async_copy(hs_hbm.at[rowv.at[2 * j + 1]], g1, sem1)
            a0.wait()
            pltpu.sync_copy(g0, macc.at[colv.at[2 * j]], add=True)
            a1.wait()
            pltpu.sync_copy(g1, macc.at[colv.at[2 * j + 1]], add=True)

    plsc.subcore_barrier()
    pltpu.sync_copy(macc.at[pl.ds(s * OPT, OPT)],
                    m_out.at[c, pl.ds(s * OPT, OPT)])

    @pl.when(s == 0)
    def _():
        pltpu.sync_copy(macc.at[pl.ds(NS * OPT, OTAIL)],
                        m_out.at[c, pl.ds(NS * OPT, OTAIL)])


def _mm_body(x_ref, w_ref, deg_ref, hs_ref):
    dinv = lax.rsqrt(deg_ref[...] + 1.0)
    h = jnp.dot(x_ref[...], w_ref[...], preferred_element_type=jnp.float32)
    hs_ref[...] = dinv * h


def _out_body(mp_ref, hs_ref, deg_ref, b_ref, out_ref):
    dinv = lax.rsqrt(deg_ref[...] + 1.0)
    m = mp_ref[0] + mp_ref[1] + hs_ref[...]
    out_ref[...] = dinv * m + b_ref[...][None, :]


def kernel(encodings, subnetwork, W, b):
    E = subnetwork.shape[1]
    row = subnetwork[0]
    col = subnetwork[1]
    rowf = jnp.concatenate([row, jnp.zeros((EPAD - E,), jnp.int32)])
    trash = TRASH + jnp.arange(EPAD - E, dtype=jnp.int32) % (NPAD - N)
    colf = jnp.concatenate([col, trash])
    rowp = rowf.reshape(TBLK, BLK)
    colp = colf.reshape(TBLK, BLK)

    deg = _deg_kernel(colf.reshape(NW, CPW, CHUNK))
    deg_col = deg[:N].reshape(N, 1)

    hs = pl.pallas_call(
        _mm_body,
        out_shape=jax.ShapeDtypeStruct((N, D), jnp.float32),
    )(encodings, W, deg_col)

    mp = _msg_kernel(hs, rowp, colp)

    out = pl.pallas_call(
        _out_body,
        out_shape=jax.ShapeDtypeStruct((N, D), jnp.float32),
    )(mp, hs, deg_col, b)
    return out


# speedup vs baseline: 1.2480x; 1.2480x over previous
import functools

import jax
import jax.numpy as jnp
from jax import lax
from jax.experimental import pallas as pl
from jax.experimental.pallas import tpu as pltpu
from jax.experimental.pallas import tpu_sc as plsc

N = 10000
D = 128
NC = 2
NS = 16
NW = NC * NS
CHUNK = 128
BLK = 128
SB = 16
C0 = 80
C1 = 80
TBLK = NS * (C0 + C1)
EPAD = TBLK * BLK
CPW = EPAD // NW // CHUNK
NPAD = 10112
TRASH = N
RPT = NPAD // NS
OPT = 624
OTAIL = N - NS * OPT

_MESH = plsc.VectorSubcoreMesh(core_axis_name="c", subcore_axis_name="s")


@functools.partial(
    pl.kernel,
    out_type=jax.ShapeDtypeStruct((NPAD,), jnp.float32),
    mesh=_MESH,
    scratch_types=[
        pltpu.VMEM((CPW, CHUNK), jnp.int32),
        pltpu.VMEM((CPW, CHUNK), jnp.int32),
        pltpu.VMEM((CHUNK,), jnp.float32),
        pltpu.VMEM((640,), jnp.float32),
        pltpu.VMEM_SHARED((NPAD,), jnp.float32),
        pltpu.SemaphoreType.DMA,
    ],
)
def _deg_kernel(col_hbm, deg_out, colva, colvb, ones, stage, dacc, sem):
    c = lax.axis_index("c")
    s = lax.axis_index("s")

    @pl.when(c == 0)
    def _():
        for i in range(640 // 16):
            stage[pl.ds(i * 16, 16)] = jnp.zeros((16,), jnp.float32)
        for i in range(CHUNK // 16):
            ones[pl.ds(i * 16, 16)] = jnp.ones((16,), jnp.float32)
        pltpu.sync_copy(stage.at[pl.ds(0, RPT)], dacc.at[pl.ds(s * RPT, RPT)])
        pltpu.sync_copy(col_hbm.at[2 * s], colva)
        pltpu.sync_copy(col_hbm.at[2 * s + 1], colvb)
        plsc.subcore_barrier()

        @pl.loop(0, CPW)
        def _(j):
            pltpu.async_copy(ones, dacc.at[colva.at[j]], sem, add=True)

        @pl.loop(0, CPW)
        def _(j):
            pltpu.async_copy(ones, dacc.at[colvb.at[j]], sem, add=True)

        @pl.loop(0, 2 * CPW)
        def _(j):
            pltpu.make_async_copy(ones, dacc.at[colva.at[0]], sem).wait()

        plsc.subcore_barrier()
        pltpu.sync_copy(dacc.at[pl.ds(s * RPT, RPT)], stage.at[pl.ds(0, RPT)])
        pltpu.sync_copy(stage.at[pl.ds(0, RPT)],
                        deg_out.at[pl.ds(s * RPT, RPT)])


@functools.partial(
    pl.kernel,
    out_type=jax.ShapeDtypeStruct((NC, N, D), jnp.float32),
    mesh=_MESH,
    scratch_types=[
        pltpu.VMEM((SB, BLK), jnp.int32),
        pltpu.VMEM((SB, BLK), jnp.int32),
        pltpu.VMEM((BLK, D), jnp.float32),
        pltpu.VMEM((BLK, D), jnp.float32),
        pltpu.VMEM_SHARED((NPAD, D), jnp.float32),
        pltpu.SemaphoreType.DMA,
        pltpu.SemaphoreType.DMA,
    ],
)
def _msg_kernel(hs_hbm, row_hbm, col_hbm, m_out, rowv, colv, g0, g1, macc,
                sem0, sem1):
    c = lax.axis_index("c")
    s = lax.axis_index("s")
    base = jnp.where(c == 0, s * C0, NS * C0 + s * C1)
    nsb = jnp.where(c == 0, C0 // SB, C1 // SB)

    @pl.loop(0, BLK)
    def _(i):
        for l in range(D // 16):
            g0[i, pl.ds(l * 16, 16)] = jnp.zeros((16,), jnp.float32)

    for t in range(RPT // BLK):
        pltpu.sync_copy(g0, macc.at[pl.ds(s * RPT + t * BLK, BLK)])
    rem = RPT - (RPT // BLK) * BLK
    pltpu.sync_copy(g0.at[pl.ds(0, rem)],
                    macc.at[pl.ds(s * RPT + RPT - rem, rem)])
    plsc.subcore_barrier()

    @pl.loop(0, nsb)
    def _(sb):
        off = pl.multiple_of(base + sb * SB, 8)
        pltpu.sync_copy(row_hbm.at[pl.ds(off, SB)], rowv)
        pltpu.sync_copy(col_hbm.at[pl.ds(off, SB)], colv)

        @pl.loop(0, SB // 2)
        def _(j):
            a0 = pltpu.async_copy(hs_hbm.at[rowv.at[2 * j]], g0, sem0)
            a1 = pltpu.async_copy(hs_hbm.at[rowv.at[2 * j + 1]], g1, sem1)
            a0.wait()
            pltpu.sync_copy(g0, macc.at[colv.at[2 * j]], add=True)
            a1.wait()
            pltpu.sync_copy(g1, macc.at[colv.at[2 * j + 1]], add=True)

    plsc.subcore_barrier()
    pltpu.sync_copy(macc.at[pl.ds(s * OPT, OPT)],
                    m_out.at[c, pl.ds(s * OPT, OPT)])

    @pl.when(s == 0)
    def _():
        pltpu.sync_copy(macc.at[pl.ds(NS * OPT, OTAIL)],
                        m_out.at[c, pl.ds(NS * OPT, OTAIL)])


def _mm_body(x_ref, w_ref, deg_ref, hs_ref):
    dinv = lax.rsqrt(deg_ref[...] + 1.0)
    h = jnp.dot(x_ref[...], w_ref[...], preferred_element_type=jnp.float32)
    hs_ref[...] = dinv * h


def _out_body(mp_ref, hs_ref, deg_ref, b_ref, out_ref):
    dinv = lax.rsqrt(deg_ref[...] + 1.0)
    m = mp_ref[0] + mp_ref[1] + hs_ref[...]
    out_ref[...] = dinv * m + b_ref[...][None, :]


def kernel(encodings, subnetwork, W, b):
    E = subnetwork.shape[1]
    row = subnetwork[0]
    col = subnetwork[1]
    rowf = jnp.concatenate([row, jnp.zeros((EPAD - E,), jnp.int32)])
    trash = TRASH + jnp.arange(EPAD - E, dtype=jnp.int32) % (NPAD - N)
    colf = jnp.concatenate([col, trash])
    BPW = TBLK // NW
    rowp = rowf.reshape(BPW, NW, BLK).swapaxes(0, 1).reshape(TBLK, BLK)
    colp3 = colf.reshape(BPW, NW, BLK).swapaxes(0, 1)
    colp = colp3.reshape(TBLK, BLK)

    deg = _deg_kernel(colp3)
    deg_col = deg[:N].reshape(N, 1)

    hs = pl.pallas_call(
        _mm_body,
        out_shape=jax.ShapeDtypeStruct((N, D), jnp.float32),
    )(encodings, W, deg_col)

    mp = _msg_kernel(hs, rowp, colp)

    out = pl.pallas_call(
        _out_body,
        out_shape=jax.ShapeDtypeStruct((N, D), jnp.float32),
    )(mp, hs, deg_col, b)
    return out


# speedup vs baseline: 2.8581x; 2.2901x over previous
import functools

import jax
import jax.numpy as jnp
from jax import lax
from jax.experimental import pallas as pl
from jax.experimental.pallas import tpu as pltpu
from jax.experimental.pallas import tpu_sc as plsc

N = 10000
D = 128
NC = 2
NS = 16
NW = NC * NS
CHUNK = 128
BLK = 128
SB = 16
C0 = 80
C1 = 80
TBLK = NS * (C0 + C1)
EPAD = TBLK * BLK
CPW = EPAD // NW // CHUNK
NPAD = 10112
TRASH = N
RPT = NPAD // NS
OPT = 624
OTAIL = N - NS * OPT

_MESH = plsc.VectorSubcoreMesh(core_axis_name="c", subcore_axis_name="s")


@functools.partial(
    pl.kernel,
    out_type=jax.ShapeDtypeStruct((NPAD,), jnp.float32),
    mesh=_MESH,
    scratch_types=[
        pltpu.VMEM((CPW, CHUNK), jnp.int32),
        pltpu.VMEM((CPW, CHUNK), jnp.int32),
        pltpu.VMEM((CHUNK,), jnp.float32),
        pltpu.VMEM((640,), jnp.float32),
        pltpu.VMEM_SHARED((NPAD,), jnp.float32),
        pltpu.SemaphoreType.DMA,
    ],
)
def _deg_kernel(col_hbm, deg_out, colva, colvb, ones, stage, dacc, sem):
    c = lax.axis_index("c")
    s = lax.axis_index("s")

    @pl.when(c == 0)
    def _():
        for i in range(640 // 16):
            stage[pl.ds(i * 16, 16)] = jnp.zeros((16,), jnp.float32)
        for i in range(CHUNK // 16):
            ones[pl.ds(i * 16, 16)] = jnp.ones((16,), jnp.float32)
        pltpu.sync_copy(stage.at[pl.ds(0, RPT)], dacc.at[pl.ds(s * RPT, RPT)])
        pltpu.sync_copy(col_hbm.at[2 * s], colva)
        pltpu.sync_copy(col_hbm.at[2 * s + 1], colvb)
        plsc.subcore_barrier()

        @pl.loop(0, CPW)
        def _(j):
            pltpu.async_copy(ones, dacc.at[colva.at[j]], sem, add=True)

        @pl.loop(0, CPW)
        def _(j):
            pltpu.async_copy(ones, dacc.at[colvb.at[j]], sem, add=True)

        @pl.loop(0, 2 * CPW)
        def _(j):
            pltpu.make_async_copy(ones, dacc.at[colva.at[0]], sem).wait()

        plsc.subcore_barrier()
        pltpu.sync_copy(dacc.at[pl.ds(s * RPT, RPT)], stage.at[pl.ds(0, RPT)])
        pltpu.sync_copy(stage.at[pl.ds(0, RPT)],
                        deg_out.at[pl.ds(s * RPT, RPT)])


@functools.partial(
    pl.kernel,
    out_type=jax.ShapeDtypeStruct((NC, N, D), jnp.float32),
    mesh=_MESH,
    scratch_types=[
        pltpu.VMEM((SB, BLK), jnp.int32),
        pltpu.VMEM((SB, BLK), jnp.int32),
        pltpu.VMEM((BLK, D), jnp.float32),
        pltpu.VMEM((BLK, D), jnp.float32),
        pltpu.VMEM_SHARED((NPAD, D), jnp.float32),
        pltpu.SemaphoreType.DMA,
        pltpu.SemaphoreType.DMA,
    ],
)
def _msg_kernel(hs_hbm, row_hbm, col_hbm, m_out, rowv, colv, g0, g1, macc,
                sem0, sem1):
    c = lax.axis_index("c")
    s = lax.axis_index("s")
    base = jnp.where(c == 0, s * C0, NS * C0 + s * C1)
    nsb = jnp.where(c == 0, C0 // SB, C1 // SB)

    @pl.loop(0, BLK)
    def _(i):
        for l in range(D // 16):
            g0[i, pl.ds(l * 16, 16)] = jnp.zeros((16,), jnp.float32)

    for t in range(RPT // BLK):
        pltpu.sync_copy(g0, macc.at[pl.ds(s * RPT + t * BLK, BLK)])
    rem = RPT - (RPT // BLK) * BLK
    pltpu.sync_copy(g0.at[pl.ds(0, rem)],
                    macc.at[pl.ds(s * RPT + RPT - rem, rem)])
    plsc.subcore_barrier()

    @pl.loop(0, nsb)
    def _(sb):
        off = pl.multiple_of(base + sb * SB, 8)
        pltpu.sync_copy(row_hbm.at[pl.ds(off, SB)], rowv)
        pltpu.sync_copy(col_hbm.at[pl.ds(off, SB)], colv)

        @pl.loop(0, SB // 2)
        def _(j):
            a0 = pltpu.async_copy(hs_hbm.at[rowv.at[2 * j]], g0, sem0)
            a1 = pltpu.async_copy(hs_hbm.at[rowv.at[2 * j + 1]], g1, sem1)
            a0.wait()
            pltpu.sync_copy(g0, macc.at[colv.at[2 * j]], add=True)
            a1.wait()
            pltpu.sync_copy(g1, macc.at[colv.at[2 * j + 1]], add=True)

    plsc.subcore_barrier()
    pltpu.sync_copy(macc.at[pl.ds(s * OPT, OPT)],
                    m_out.at[c, pl.ds(s * OPT, OPT)])

    @pl.when(s == 0)
    def _():
        pltpu.sync_copy(macc.at[pl.ds(NS * OPT, OTAIL)],
                        m_out.at[c, pl.ds(NS * OPT, OTAIL)])


def _mm_body(x_ref, w_ref, deg_ref, hs_ref):
    dinv = lax.rsqrt(deg_ref[...] + 1.0)
    h = jnp.dot(x_ref[...], w_ref[...], preferred_element_type=jnp.float32)
    hs_ref[...] = dinv * h


def _out_body(mp_ref, hs_ref, deg_ref, b_ref, out_ref):
    dinv = lax.rsqrt(deg_ref[...] + 1.0)
    m = mp_ref[0] + mp_ref[1] + hs_ref[...]
    out_ref[...] = dinv * m + b_ref[...][None, :]


def kernel(encodings, subnetwork, W, b):
    E = subnetwork.shape[1]
    row = subnetwork[0]
    col = subnetwork[1]
    pad = jnp.arange(EPAD - E, dtype=jnp.int32)
    rowf = jnp.concatenate([row, pad % N])
    colf = jnp.concatenate([col, TRASH + pad % (NPAD - N)])
    BPW = TBLK // NW
    rowp = rowf.reshape(BPW, NW, BLK).swapaxes(0, 1).reshape(TBLK, BLK)
    colp3 = colf.reshape(BPW, NW, BLK).swapaxes(0, 1)
    colp = colp3.reshape(TBLK, BLK)

    deg = _deg_kernel(colp3)
    deg_col = deg[:N].reshape(N, 1)

    hs = pl.pallas_call(
        _mm_body,
        out_shape=jax.ShapeDtypeStruct((N, D), jnp.float32),
    )(encodings, W, deg_col)

    mp = _msg_kernel(hs, rowp, colp)

    out = pl.pallas_call(
        _out_body,
        out_shape=jax.ShapeDtypeStruct((N, D), jnp.float32),
    )(mp, hs, deg_col, b)
    return out


# speedup vs baseline: 3.4342x; 1.2016x over previous
import functools

import jax
import jax.numpy as jnp
from jax import lax
from jax.experimental import pallas as pl
from jax.experimental.pallas import tpu as pltpu
from jax.experimental.pallas import tpu_sc as plsc

N = 10000
D = 128
NC = 2
NS = 16
NW = NC * NS
CHUNK = 128
BLK = 112
SB = 16
NBUF = 3
BPT = 96
TBLK = NW * BPT
EPAD = TBLK * BLK
CPW = EPAD // NW // CHUNK
NPAD = 10112
TRASH = N
RPT = NPAD // NS
OPT = 624
OTAIL = N - NS * OPT

_MESH = plsc.VectorSubcoreMesh(core_axis_name="c", subcore_axis_name="s")


@functools.partial(
    pl.kernel,
    out_type=jax.ShapeDtypeStruct((NPAD,), jnp.float32),
    mesh=_MESH,
    scratch_types=[
        pltpu.VMEM((CPW, CHUNK), jnp.int32),
        pltpu.VMEM((CPW, CHUNK), jnp.int32),
        pltpu.VMEM((CHUNK,), jnp.float32),
        pltpu.VMEM((640,), jnp.float32),
        pltpu.VMEM_SHARED((NPAD,), jnp.float32),
        pltpu.SemaphoreType.DMA,
    ],
)
def _deg_kernel(col_hbm, deg_out, colva, colvb, ones, stage, dacc, sem):
    c = lax.axis_index("c")
    s = lax.axis_index("s")

    @pl.when(c == 0)
    def _():
        for i in range(640 // 16):
            stage[pl.ds(i * 16, 16)] = jnp.zeros((16,), jnp.float32)
        for i in range(CHUNK // 16):
            ones[pl.ds(i * 16, 16)] = jnp.ones((16,), jnp.float32)
        pltpu.sync_copy(stage.at[pl.ds(0, RPT)], dacc.at[pl.ds(s * RPT, RPT)])
        pltpu.sync_copy(col_hbm.at[2 * s], colva)
        pltpu.sync_copy(col_hbm.at[2 * s + 1], colvb)
        plsc.subcore_barrier()

        @pl.loop(0, CPW)
        def _(j):
            pltpu.async_copy(ones, dacc.at[colva.at[j]], sem, add=True)

        @pl.loop(0, CPW)
        def _(j):
            pltpu.async_copy(ones, dacc.at[colvb.at[j]], sem, add=True)

        @pl.loop(0, 2 * CPW)
        def _(j):
            pltpu.make_async_copy(ones, dacc.at[colva.at[0]], sem).wait()

        plsc.subcore_barrier()
        pltpu.sync_copy(dacc.at[pl.ds(s * RPT, RPT)], stage.at[pl.ds(0, RPT)])
        pltpu.sync_copy(stage.at[pl.ds(0, RPT)],
                        deg_out.at[pl.ds(s * RPT, RPT)])


@functools.partial(
    pl.kernel,
    out_type=jax.ShapeDtypeStruct((NC, N, D), jnp.float32),
    mesh=_MESH,
    scratch_types=[
        pltpu.VMEM((SB, BLK), jnp.int32),
        pltpu.VMEM((SB, BLK), jnp.int32),
        [pltpu.VMEM((BLK, D), jnp.float32) for _ in range(NBUF)],
        pltpu.VMEM_SHARED((NPAD, D), jnp.float32),
        [pltpu.SemaphoreType.DMA for _ in range(NBUF)],
        [pltpu.SemaphoreType.DMA for _ in range(NBUF)],
    ],
)
def _msg_kernel(hs_hbm, row_hbm, col_hbm, m_out, rowv, colv, bufs, macc,
                gsem, ssem):
    c = lax.axis_index("c")
    s = lax.axis_index("s")
    wid = c * NS + s
    base = wid * BPT

    @pl.loop(0, BLK)
    def _(i):
        for l in range(D // 16):
            bufs[0][i, pl.ds(l * 16, 16)] = jnp.zeros((16,), jnp.float32)

    for t in range(RPT // BLK):
        pltpu.sync_copy(bufs[0], macc.at[pl.ds(s * RPT + t * BLK, BLK)])
    rem = RPT - (RPT // BLK) * BLK
    pltpu.sync_copy(bufs[0].at[pl.ds(0, rem)],
                    macc.at[pl.ds(s * RPT + RPT - rem, rem)])
    plsc.subcore_barrier()

    @pl.loop(0, BPT // SB)
    def _(sb):
        off = pl.multiple_of(base + sb * SB, 8)
        pltpu.sync_copy(row_hbm.at[pl.ds(off, SB)], rowv)
        pltpu.sync_copy(col_hbm.at[pl.ds(off, SB)], colv)

        gd = {}
        sd = {}
        waited = set()

        def fire_gather(j):
            bb = j % NBUF
            gd[j] = pltpu.async_copy(hs_hbm.at[rowv.at[j]], bufs[bb], gsem[bb])

        fire_gather(0)
        fire_gather(1)
        for i in range(SB):
            j = i + 2
            if j < SB:
                if (j - NBUF) in sd:
                    sd[j - NBUF].wait()
                    waited.add(j - NBUF)
                fire_gather(j)
            gd[i].wait()
            sd[i] = pltpu.async_copy(bufs[i % NBUF], macc.at[colv.at[i]],
                                     ssem[i % NBUF], add=True)
        for i in range(SB):
            if i not in waited:
                sd[i].wait()

    plsc.subcore_barrier()
    pltpu.sync_copy(macc.at[pl.ds(s * OPT, OPT)],
                    m_out.at[c, pl.ds(s * OPT, OPT)])

    @pl.when(s == 0)
    def _():
        pltpu.sync_copy(macc.at[pl.ds(NS * OPT, OTAIL)],
                        m_out.at[c, pl.ds(NS * OPT, OTAIL)])


def _mm_body(x_ref, w_ref, deg_ref, hs_ref):
    dinv = lax.rsqrt(deg_ref[...] + 1.0)
    h = jnp.dot(x_ref[...], w_ref[...], preferred_element_type=jnp.float32)
    hs_ref[...] = dinv * h


def _out_body(mp_ref, hs_ref, deg_ref, b_ref, out_ref):
    dinv = lax.rsqrt(deg_ref[...] + 1.0)
    m = mp_ref[0] + mp_ref[1] + hs_ref[...]
    out_ref[...] = dinv * m + b_ref[...][None, :]


def kernel(encodings, subnetwork, W, b):
    E = subnetwork.shape[1]
    row = subnetwork[0]
    col = subnetwork[1]
    pad = jnp.arange(EPAD - E, dtype=jnp.int32)
    rowf = jnp.concatenate([row, pad % N])
    colf = jnp.concatenate([col, TRASH + pad % (NPAD - N)])
    BPW = TBLK // NW
    rowp = rowf.reshape(BPW, NW, BLK).swapaxes(0, 1).reshape(TBLK, BLK)
    colp3 = colf.reshape(BPW, NW, BLK).swapaxes(0, 1)
    colp = colp3.reshape(TBLK, BLK)

    deg = _deg_kernel(colp3.reshape(NW, CPW, CHUNK))
    deg_col = deg[:N].reshape(N, 1)

    hs = pl.pallas_call(
        _mm_body,
        out_shape=jax.ShapeDtypeStruct((N, D), jnp.float32),
    )(encodings, W, deg_col)

    mp = _msg_kernel(hs, rowp, colp)

    out = pl.pallas_call(
        _out_body,
        out_shape=jax.ShapeDtypeStruct((N, D), jnp.float32),
    )(mp, hs, deg_col, b)
    return out


# speedup vs baseline: 3.5427x; 1.0316x over previous
import functools

import jax
import jax.numpy as jnp
from jax import lax
from jax.experimental import pallas as pl
from jax.experimental.pallas import tpu as pltpu
from jax.experimental.pallas import tpu_sc as plsc

N = 10000
D = 128
NC = 2
NS = 16
NW = NC * NS
CHUNK = 128
BLK = 112
SB = 24
NBUF = 3
BPT = 96
TBLK = NW * BPT
EPAD = TBLK * BLK
CPW = EPAD // NW // CHUNK
NPAD = 10112
TRASH = N
RPT = NPAD // NS
OPT = 624
OTAIL = N - NS * OPT

_MESH = plsc.VectorSubcoreMesh(core_axis_name="c", subcore_axis_name="s")


@functools.partial(
    pl.kernel,
    out_type=jax.ShapeDtypeStruct((NPAD,), jnp.float32),
    mesh=_MESH,
    scratch_types=[
        pltpu.VMEM((CPW, CHUNK), jnp.int32),
        pltpu.VMEM((CPW, CHUNK), jnp.int32),
        pltpu.VMEM((CHUNK,), jnp.float32),
        pltpu.VMEM((640,), jnp.float32),
        pltpu.VMEM_SHARED((NPAD,), jnp.float32),
        pltpu.SemaphoreType.DMA,
    ],
)
def _deg_kernel(col_hbm, deg_out, colva, colvb, ones, stage, dacc, sem):
    c = lax.axis_index("c")
    s = lax.axis_index("s")

    @pl.when(c == 0)
    def _():
        for i in range(640 // 16):
            stage[pl.ds(i * 16, 16)] = jnp.zeros((16,), jnp.float32)
        for i in range(CHUNK // 16):
            ones[pl.ds(i * 16, 16)] = jnp.ones((16,), jnp.float32)
        pltpu.sync_copy(stage.at[pl.ds(0, RPT)], dacc.at[pl.ds(s * RPT, RPT)])
        pltpu.sync_copy(col_hbm.at[2 * s], colva)
        pltpu.sync_copy(col_hbm.at[2 * s + 1], colvb)
        plsc.subcore_barrier()

        @pl.loop(0, CPW)
        def _(j):
            pltpu.async_copy(ones, dacc.at[colva.at[j]], sem, add=True)

        @pl.loop(0, CPW)
        def _(j):
            pltpu.async_copy(ones, dacc.at[colvb.at[j]], sem, add=True)

        @pl.loop(0, 2 * CPW)
        def _(j):
            pltpu.make_async_copy(ones, dacc.at[colva.at[0]], sem).wait()

        plsc.subcore_barrier()
        pltpu.sync_copy(dacc.at[pl.ds(s * RPT, RPT)], stage.at[pl.ds(0, RPT)])
        pltpu.sync_copy(stage.at[pl.ds(0, RPT)],
                        deg_out.at[pl.ds(s * RPT, RPT)])


@functools.partial(
    pl.kernel,
    out_type=jax.ShapeDtypeStruct((NC, N, D), jnp.float32),
    mesh=_MESH,
    scratch_types=[
        pltpu.VMEM((SB, BLK), jnp.int32),
        pltpu.VMEM((SB, BLK), jnp.int32),
        [pltpu.VMEM((BLK, D), jnp.float32) for _ in range(NBUF)],
        pltpu.VMEM_SHARED((NPAD, D), jnp.float32),
        [pltpu.SemaphoreType.DMA for _ in range(NBUF)],
        [pltpu.SemaphoreType.DMA for _ in range(NBUF)],
    ],
)
def _msg_kernel(hs_hbm, row_hbm, col_hbm, m_out, rowv, colv, bufs, macc,
                gsem, ssem):
    c = lax.axis_index("c")
    s = lax.axis_index("s")
    wid = c * NS + s
    base = wid * BPT

    @pl.loop(0, BLK)
    def _(i):
        for l in range(D // 16):
            bufs[0][i, pl.ds(l * 16, 16)] = jnp.zeros((16,), jnp.float32)

    for t in range(RPT // BLK):
        pltpu.sync_copy(bufs[0], macc.at[pl.ds(s * RPT + t * BLK, BLK)])
    rem = RPT - (RPT // BLK) * BLK
    pltpu.sync_copy(bufs[0].at[pl.ds(0, rem)],
                    macc.at[pl.ds(s * RPT + RPT - rem, rem)])
    plsc.subcore_barrier()

    @pl.loop(0, BPT // SB)
    def _(sb):
        off = pl.multiple_of(base + sb * SB, 8)
        pltpu.sync_copy(row_hbm.at[pl.ds(off, SB)], rowv)
        pltpu.sync_copy(col_hbm.at[pl.ds(off, SB)], colv)

        gd = {}
        sd = {}
        waited = set()

        def fire_gather(j):
            bb = j % NBUF
            gd[j] = pltpu.async_copy(hs_hbm.at[rowv.at[j]], bufs[bb], gsem[bb])

        fire_gather(0)
        fire_gather(1)
        for i in range(SB):
            j = i + 2
            if j < SB:
                if (j - NBUF) in sd:
                    sd[j - NBUF].wait()
                    waited.add(j - NBUF)
                fire_gather(j)
            gd[i].wait()
            sd[i] = pltpu.async_copy(bufs[i % NBUF], macc.at[colv.at[i]],
                                     ssem[i % NBUF], add=True)
        for i in range(SB):
            if i not in waited:
                sd[i].wait()

    plsc.subcore_barrier()
    pltpu.sync_copy(macc.at[pl.ds(s * OPT, OPT)],
                    m_out.at[c, pl.ds(s * OPT, OPT)])

    @pl.when(s == 0)
    def _():
        pltpu.sync_copy(macc.at[pl.ds(NS * OPT, OTAIL)],
                        m_out.at[c, pl.ds(NS * OPT, OTAIL)])


def _mm_body(x_ref, w_ref, deg_ref, hs_ref):
    dinv = lax.rsqrt(deg_ref[...] + 1.0)
    h = jnp.dot(x_ref[...], w_ref[...], preferred_element_type=jnp.float32)
    hs_ref[...] = dinv * h


def _out_body(mp_ref, hs_ref, deg_ref, b_ref, out_ref):
    dinv = lax.rsqrt(deg_ref[...] + 1.0)
    m = mp_ref[0] + mp_ref[1] + hs_ref[...]
    out_ref[...] = dinv * m + b_ref[...][None, :]


def kernel(encodings, subnetwork, W, b):
    E = subnetwork.shape[1]
    row = subnetwork[0]
    col = subnetwork[1]
    pad = jnp.arange(EPAD - E, dtype=jnp.int32)
    rowf = jnp.concatenate([row, pad % N])
    colf = jnp.concatenate([col, TRASH + pad % (NPAD - N)])
    BPW = TBLK // NW
    rowp = rowf.reshape(BPW, NW, BLK).swapaxes(0, 1).reshape(TBLK, BLK)
    colp3 = colf.reshape(BPW, NW, BLK).swapaxes(0, 1)
    colp = colp3.reshape(TBLK, BLK)

    deg = _deg_kernel(colp3.reshape(NW, CPW, CHUNK))
    deg_col = deg[:N].reshape(N, 1)

    hs = pl.pallas_call(
        _mm_body,
        out_shape=jax.ShapeDtypeStruct((N, D), jnp.float32),
    )(encodings, W, deg_col)

    mp = _msg_kernel(hs, rowp, colp)

    out = pl.pallas_call(
        _out_body,
        out_shape=jax.ShapeDtypeStruct((N, D), jnp.float32),
    )(mp, hs, deg_col, b)
    return out
